# Initial kernel scaffold; baseline (speedup 1.0000x reference)
import functools

import jax
import jax.numpy as jnp
from jax import lax
from jax.experimental import pallas as pl
from jax.experimental.pallas import tpu as pltpu
from jax.experimental.pallas import tpu_sc as plsc

N_NODES = 10000
N_EDGES = 320000
D_FEAT = 128
H1 = 256
H2 = 256
D_OUT = 128

NC = 2
NS = 16
EB = 80
RPT = N_NODES // NS


def _sc_mesh():
    return plsc.VectorSubcoreMesh(
        core_axis_name="c", subcore_axis_name="s", num_cores=NC, num_subcores=NS
    )


def _deg_body(src_hbm, dst_hbm, e1_hbm, e2_hbm, zero_hbm, outa, outb,
              sidx, didx, e1v, e2v, zbuf, acc):
    c = lax.axis_index("c")
    s = lax.axis_index("s")
    pltpu.sync_copy(zero_hbm, zbuf)
    pltpu.sync_copy(zbuf, acc.at[pl.ds(s * RPT, RPT)])
    pltpu.sync_copy(e1_hbm, e1v)
    pltpu.sync_copy(e2_hbm, e2v)
    plsc.subcore_barrier()
    epw = N_EDGES // (NC * NS)

    def it(i, carry):
        base = (c * NS + s) * epw + i * EB
        pltpu.sync_copy(src_hbm.at[pl.ds(base, EB)], sidx)
        pltpu.sync_copy(dst_hbm.at[pl.ds(base, EB)], didx)
        pltpu.sync_copy(e1v, acc.at[sidx], add=True)
        pltpu.sync_copy(e2v, acc.at[didx], add=True)
        return carry

    lax.fori_loop(0, epw // EB, it, 0)
    plsc.subcore_barrier()
    pltpu.sync_copy(acc.at[pl.ds(s * RPT, RPT)], zbuf)

    @pl.when(c == 0)
    def _():
        pltpu.sync_copy(zbuf, outa.at[pl.ds(s * RPT, RPT)])

    @pl.when(c == 1)
    def _():
        pltpu.sync_copy(zbuf, outb.at[pl.ds(s * RPT, RPT)])


_deg_call = pl.kernel(
    _deg_body,
    out_type=[
        jax.ShapeDtypeStruct((N_NODES, 2), jnp.float32),
        jax.ShapeDtypeStruct((N_NODES, 2), jnp.float32),
    ],
    mesh=_sc_mesh(),
    scratch_types=[
        pltpu.VMEM((EB,), jnp.int32),
        pltpu.VMEM((EB,), jnp.int32),
        pltpu.VMEM((EB, 2), jnp.float32),
        pltpu.VMEM((EB, 2), jnp.float32),
        pltpu.VMEM((RPT, 2), jnp.float32),
        pltpu.VMEM_SHARED((N_NODES, 2), jnp.float32),
    ],
)


def _make_agg(chunk):
    epw = N_EDGES // NS
    nb = epw // EB

    def body(hsa, hsb, src_hbm, dst_hbm, zero_hbm, outa, outb,
             sidx, didx, rows, obuf, acc, sem):
        c = lax.axis_index("c")
        s = lax.axis_index("s")
        pltpu.sync_copy(zero_hbm, obuf)
        pltpu.sync_copy(obuf, acc.at[pl.ds(s * RPT, RPT)])
        plsc.subcore_barrier()

        def make_it(hs):
            def it(i, carry):
                base = s * epw + i * EB
                pltpu.sync_copy(src_hbm.at[pl.ds(base, EB)], sidx)
                pltpu.sync_copy(dst_hbm.at[pl.ds(base, EB)], didx)
                pltpu.async_copy(hs.at[sidx], rows, sem).wait()
                pltpu.sync_copy(rows, acc.at[didx], add=True)
                return carry
            return it

        @pl.when(c == 0)
        def _():
            lax.fori_loop(0, nb, make_it(hsa), 0)

        @pl.when(c == 1)
        def _():
            lax.fori_loop(0, nb, make_it(hsb), 0)

        plsc.subcore_barrier()
        pltpu.sync_copy(acc.at[pl.ds(s * RPT, RPT)], obuf)

        @pl.when(c == 0)
        def _():
            pltpu.sync_copy(obuf, outa.at[pl.ds(s * RPT, RPT)])

        @pl.when(c == 1)
        def _():
            pltpu.sync_copy(obuf, outb.at[pl.ds(s * RPT, RPT)])

    return pl.kernel(
        body,
        out_type=[
            jax.ShapeDtypeStruct((N_NODES, chunk), jnp.float32),
            jax.ShapeDtypeStruct((N_NODES, chunk), jnp.float32),
        ],
        mesh=_sc_mesh(),
        scratch_types=[
            pltpu.VMEM((EB,), jnp.int32),
            pltpu.VMEM((EB,), jnp.int32),
            pltpu.VMEM((EB, chunk), jnp.float32),
            pltpu.VMEM((RPT, chunk), jnp.float32),
            pltpu.VMEM_SHARED((N_NODES, chunk), jnp.float32),
            pltpu.SemaphoreType.DMA,
        ],
    )


_agg64 = _make_agg(64)
_agg128 = _make_agg(128)


_RB = 1000
_NB = N_NODES // _RB


def _rows(ref, i):
    return ref[pl.ds(i * _RB, _RB), :]


def _tc1_body(aa_ref, ab_ref, nd_ref, ns_ref, w_ref, b_ref, oa_ref, ob_ref):
    i = pl.program_id(0)
    nd = _rows(nd_ref, i)
    agg = jnp.concatenate([aa_ref[...], ab_ref[...]], axis=1) * nd
    h = jnp.tanh(
        jnp.dot(agg, w_ref[...], preferred_element_type=jnp.float32) + b_ref[...]
    )
    h = h * _rows(ns_ref, i)
    oa_ref[...] = h[:, : H1 // 2]
    ob_ref[...] = h[:, H1 // 2:]


def _tc2_body(aa_ref, ab_ref, nd_ref, ns_ref, w_ref, b_ref, wm_ref,
              oa_ref, ob_ref):
    i = pl.program_id(0)
    nd = _rows(nd_ref, i)
    agg = jnp.concatenate([aa_ref[...], ab_ref[...]], axis=1) * nd
    h = jax.nn.relu(
        jnp.dot(agg, w_ref[...], preferred_element_type=jnp.float32) + b_ref[...]
    )
    h = h * _rows(ns_ref, i)
    hp = jnp.dot(h, wm_ref[...], preferred_element_type=jnp.float32)
    oa_ref[...] = hp[:, : D_OUT // 2]
    ob_ref[...] = hp[:, D_OUT // 2:]


def _tc3_body(aa_ref, ab_ref, nd_ref, b_ref, eps_ref, z_ref, zm_ref):
    i = pl.program_id(0)
    nd = _rows(nd_ref, i)
    zm = jnp.concatenate([aa_ref[...], ab_ref[...]], axis=1) * nd + b_ref[...]
    zm_ref[...] = zm
    z_ref[...] = zm + jnp.exp(zm) * eps_ref[...]


def _full(shape):
    return pl.BlockSpec(shape, lambda i: (0, 0))


def _blk(cols):
    return pl.BlockSpec((_RB, cols), lambda i: (i, 0))


def _tc1(aa, ab, nd, ns, w, b):
    return pl.pallas_call(
        _tc1_body,
        grid=(_NB,),
        in_specs=[
            _blk(D_FEAT // 2), _blk(D_FEAT // 2),
            _full((N_NODES, 1)), _full((N_NODES, 1)),
            _full((D_FEAT, H1)), _full((1, H1)),
        ],
        out_specs=[_blk(H1 // 2), _blk(H1 // 2)],
        out_shape=[
            jax.ShapeDtypeStruct((N_NODES, H1 // 2), jnp.float32),
            jax.ShapeDtypeStruct((N_NODES, H1 // 2), jnp.float32),
        ],
    )(aa, ab, nd, ns, w, b)


def _tc2(aa, ab, nd, ns, w, b, wm):
    return pl.pallas_call(
        _tc2_body,
        grid=(_NB,),
        in_specs=[
            _blk(H1 // 2), _blk(H1 // 2),
            _full((N_NODES, 1)), _full((N_NODES, 1)),
            _full((H1, H2)), _full((1, H2)), _full((H2, D_OUT)),
        ],
        out_specs=[_blk(D_OUT // 2), _blk(D_OUT // 2)],
        out_shape=[
            jax.ShapeDtypeStruct((N_NODES, D_OUT // 2), jnp.float32),
            jax.ShapeDtypeStruct((N_NODES, D_OUT // 2), jnp.float32),
        ],
    )(aa, ab, nd, ns, w, b, wm)


def _tc3(aa, ab, nd, b, eps):
    return pl.pallas_call(
        _tc3_body,
        grid=(_NB,),
        in_specs=[
            _blk(D_OUT // 2), _blk(D_OUT // 2),
            _full((N_NODES, 1)), _full((1, D_OUT)), _blk(D_OUT),
        ],
        out_specs=[_blk(D_OUT), _blk(D_OUT)],
        out_shape=[
            jax.ShapeDtypeStruct((N_NODES, D_OUT), jnp.float32),
            jax.ShapeDtypeStruct((N_NODES, D_OUT), jnp.float32),
        ],
    )(aa, ab, nd, b, eps)


def kernel(x, edge_index, W1, b1, W2, b2, Wm, bm):
    src = edge_index[0]
    dst = edge_index[1]

    e1 = jnp.tile(jnp.array([[1.0, 0.0]], jnp.float32), (EB, 1))
    e2 = jnp.tile(jnp.array([[0.0, 1.0]], jnp.float32), (EB, 1))
    z2 = jnp.zeros((RPT, 2), jnp.float32)
    z64 = jnp.zeros((RPT, 64), jnp.float32)
    z128 = jnp.zeros((RPT, 128), jnp.float32)

    dega, degb = _deg_call(src, dst, e1, e2, z2)
    deg = dega + degb
    ns = lax.rsqrt(jnp.clip(deg[:, :1], 1.0, None))
    nd = lax.rsqrt(jnp.clip(deg[:, 1:], 1.0, None))

    xs = x * ns
    a1a, a1b = _agg64(xs[:, :64], xs[:, 64:], src, dst, z64)
    h1a, h1b = _tc1(a1a, a1b, nd, ns, W1, b1.reshape(1, H1))
    a2a, a2b = _agg128(h1a, h1b, src, dst, z128)
    h2a, h2b = _tc2(a2a, a2b, nd, ns, W2, b2.reshape(1, H2), Wm)
    a3a, a3b = _agg64(h2a, h2b, src, dst, z64)

    eps = jax.random.normal(jax.random.key(42), (N_NODES, D_OUT), jnp.float32)
    z, zm = _tc3(a3a, a3b, nd, bm.reshape(1, D_OUT), eps)
    return (z, zm, zm)


# speedup vs baseline: 3.5819x; 3.5819x over previous
import jax
import jax.numpy as jnp
from jax import lax
from jax.experimental import pallas as pl
from jax.experimental.pallas import tpu as pltpu
from jax.experimental.pallas import tpu_sc as plsc

N_NODES = 10000
N_EDGES = 320000
D_FEAT = 128
H1 = 256
H2 = 256
D_OUT = 128

NC = 2
NS = 16
EB = 80
NPAD = 10240
RPT = NPAD // NS
CHUNK_ROWS = 80
NCH = RPT // CHUNK_ROWS


def _sc_mesh():
    return plsc.VectorSubcoreMesh(
        core_axis_name="c", subcore_axis_name="s", num_cores=NC, num_subcores=NS
    )


def _zero_acc(zero_hbm, zbuf, acc, s):
    pltpu.sync_copy(zero_hbm, zbuf)

    def z(j, carry):
        pltpu.sync_copy(zbuf, acc.at[pl.ds(s * RPT + j * CHUNK_ROWS, CHUNK_ROWS)])
        return carry

    lax.fori_loop(0, NCH, z, 0)


def _copy_out(acc, obuf, out_hbm, s):
    def co(j, carry):
        r = s * RPT + j * CHUNK_ROWS
        pltpu.sync_copy(acc.at[pl.ds(r, CHUNK_ROWS)], obuf)
        pltpu.sync_copy(obuf, out_hbm.at[pl.ds(r, CHUNK_ROWS)])
        return carry

    lax.fori_loop(0, NCH, co, 0)


def _deg_body(src_hbm, dst_hbm, ones_hbm, zero_hbm, outa, outb,
              didx, onesv, obuf, acc):
    c = lax.axis_index("c")
    s = lax.axis_index("s")
    _zero_acc(zero_hbm, obuf, acc, s)
    pltpu.sync_copy(ones_hbm, onesv)
    plsc.subcore_barrier()
    epw = N_EDGES // NS

    def make_it(idx_hbm):
        def it(i, carry):
            base = s * epw + i * EB
            pltpu.sync_copy(idx_hbm.at[pl.ds(base, EB)], didx)
            pltpu.sync_copy(onesv, acc.at[didx], add=True)
            return carry
        return it

    @pl.when(c == 0)
    def _():
        lax.fori_loop(0, epw // EB, make_it(dst_hbm), 0)

    @pl.when(c == 1)
    def _():
        lax.fori_loop(0, epw // EB, make_it(src_hbm), 0)

    plsc.subcore_barrier()

    @pl.when(c == 0)
    def _():
        _copy_out(acc, obuf, outa, s)

    @pl.when(c == 1)
    def _():
        _copy_out(acc, obuf, outb, s)


_deg_call = pl.kernel(
    _deg_body,
    out_type=[
        jax.ShapeDtypeStruct((NPAD, 128), jnp.float32),
        jax.ShapeDtypeStruct((NPAD, 128), jnp.float32),
    ],
    mesh=_sc_mesh(),
    scratch_types=[
        pltpu.VMEM((EB,), jnp.int32),
        pltpu.VMEM((EB, 128), jnp.float32),
        pltpu.VMEM((CHUNK_ROWS, 128), jnp.float32),
        pltpu.VMEM_SHARED((NPAD, 128), jnp.float32),
    ],
)


def _agg_body(hsa, hsb, src_hbm, dst_hbm, zero_hbm, outa, outb,
              sidx, didx, rows, obuf, acc, sem):
    c = lax.axis_index("c")
    s = lax.axis_index("s")
    _zero_acc(zero_hbm, obuf, acc, s)
    plsc.subcore_barrier()
    epw = N_EDGES // NS // 2
    nb = epw // EB

    def make_it(hs, ebase):
        def it(i, carry):
            base = ebase + s * epw + i * EB
            pltpu.sync_copy(src_hbm.at[pl.ds(base, EB)], sidx)
            pltpu.sync_copy(dst_hbm.at[pl.ds(base, EB)], didx)
            pltpu.async_copy(hs.at[sidx], rows, sem).wait()
            pltpu.sync_copy(rows, acc.at[didx], add=True)
            return carry
        return it

    @pl.when(c == 0)
    def _():
        lax.fori_loop(0, nb, make_it(hsa, 0), 0)

    @pl.when(c == 1)
    def _():
        lax.fori_loop(0, nb, make_it(hsb, N_EDGES // 2), 0)

    plsc.subcore_barrier()

    @pl.when(c == 0)
    def _():
        _copy_out(acc, obuf, outa, s)

    @pl.when(c == 1)
    def _():
        _copy_out(acc, obuf, outb, s)


_agg = pl.kernel(
    _agg_body,
    out_type=[
        jax.ShapeDtypeStruct((NPAD, 128), jnp.float32),
        jax.ShapeDtypeStruct((NPAD, 128), jnp.float32),
    ],
    mesh=_sc_mesh(),
    scratch_types=[
        pltpu.VMEM((EB,), jnp.int32),
        pltpu.VMEM((EB,), jnp.int32),
        pltpu.VMEM((EB, 128), jnp.float32),
        pltpu.VMEM((CHUNK_ROWS, 128), jnp.float32),
        pltpu.VMEM_SHARED((NPAD, 128), jnp.float32),
        pltpu.SemaphoreType.DMA,
    ],
)


_RB = 1024
_NB = NPAD // _RB


def _rows(ref, i):
    return ref[pl.ds(i * _RB, _RB), :]


def _tc1_body(aa_ref, ab_ref, nd_ref, ns_ref, w_ref, b_ref, oa_ref, ob_ref):
    i = pl.program_id(0)
    nd = _rows(nd_ref, i)
    agg = (aa_ref[...] + ab_ref[...]) * nd
    h = jnp.tanh(
        jnp.dot(agg, w_ref[...], preferred_element_type=jnp.float32) + b_ref[...]
    )
    h = h * _rows(ns_ref, i)
    oa_ref[...] = h[:, : H1 // 2]
    ob_ref[...] = h[:, H1 // 2:]


def _tc2_body(aa0_ref, aa1_ref, ab0_ref, ab1_ref, nd_ref, ns_ref, w_ref,
              b_ref, wm_ref, o_ref):
    i = pl.program_id(0)
    nd = _rows(nd_ref, i)
    agg = jnp.concatenate(
        [aa0_ref[...] + aa1_ref[...], ab0_ref[...] + ab1_ref[...]], axis=1
    ) * nd
    h = jax.nn.relu(
        jnp.dot(agg, w_ref[...], preferred_element_type=jnp.float32) + b_ref[...]
    )
    h = h * _rows(ns_ref, i)
    o_ref[...] = jnp.dot(h, wm_ref[...], preferred_element_type=jnp.float32)


def _tc3_body(aa_ref, ab_ref, nd_ref, b_ref, eps_ref, z_ref, zm_ref):
    i = pl.program_id(0)
    nd = _rows(nd_ref, i)
    zm = (aa_ref[...] + ab_ref[...]) * nd + b_ref[...]
    zm_ref[...] = zm
    z_ref[...] = zm + jnp.exp(zm) * eps_ref[...]


def _full(shape):
    return pl.BlockSpec(shape, lambda i: (0, 0))


def _blk(cols):
    return pl.BlockSpec((_RB, cols), lambda i: (i, 0))


def _tc1(aa, ab, nd, ns, w, b):
    return pl.pallas_call(
        _tc1_body,
        grid=(_NB,),
        in_specs=[
            _blk(D_FEAT), _blk(D_FEAT),
            _full((NPAD, 1)), _full((NPAD, 1)),
            _full((D_FEAT, H1)), _full((1, H1)),
        ],
        out_specs=[_blk(H1 // 2), _blk(H1 // 2)],
        out_shape=[
            jax.ShapeDtypeStruct((NPAD, H1 // 2), jnp.float32),
            jax.ShapeDtypeStruct((NPAD, H1 // 2), jnp.float32),
        ],
    )(aa, ab, nd, ns, w, b)


def _tc2(aa0, aa1, ab0, ab1, nd, ns, w, b, wm):
    return pl.pallas_call(
        _tc2_body,
        grid=(_NB,),
        in_specs=[
            _blk(H1 // 2), _blk(H1 // 2), _blk(H1 // 2), _blk(H1 // 2),
            _full((NPAD, 1)), _full((NPAD, 1)),
            _full((H1, H2)), _full((1, H2)), _full((H2, D_OUT)),
        ],
        out_specs=_blk(D_OUT),
        out_shape=jax.ShapeDtypeStruct((NPAD, D_OUT), jnp.float32),
    )(aa0, aa1, ab0, ab1, nd, ns, w, b, wm)


def _tc3(aa, ab, nd, b, eps):
    return pl.pallas_call(
        _tc3_body,
        grid=(_NB,),
        in_specs=[
            _blk(D_OUT), _blk(D_OUT),
            _full((NPAD, 1)), _full((1, D_OUT)), _blk(D_OUT),
        ],
        out_specs=[_blk(D_OUT), _blk(D_OUT)],
        out_shape=[
            jax.ShapeDtypeStruct((NPAD, D_OUT), jnp.float32),
            jax.ShapeDtypeStruct((NPAD, D_OUT), jnp.float32),
        ],
    )(aa, ab, nd, b, eps)


def kernel(x, edge_index, W1, b1, W2, b2, Wm, bm):
    src = edge_index[0]
    dst = edge_index[1]

    ones = jnp.ones((EB, 128), jnp.float32)
    zrows = jnp.zeros((CHUNK_ROWS, 128), jnp.float32)

    degi, dego = _deg_call(src, dst, ones, zrows)
    ns = lax.rsqrt(jnp.clip(dego[:, :1], 1.0, None))
    nd = lax.rsqrt(jnp.clip(degi[:, :1], 1.0, None))

    xs = jnp.pad(x * ns[:N_NODES], ((0, NPAD - N_NODES), (0, 0)))
    a1a, a1b = _agg(xs, xs, src, dst, zrows)
    h1a, h1b = _tc1(a1a, a1b, nd, ns, W1, b1.reshape(1, H1))
    a2aa, a2ab = _agg(h1a, h1a, src, dst, zrows)
    a2ba, a2bb = _agg(h1b, h1b, src, dst, zrows)
    h2p = _tc2(a2aa, a2ab, a2ba, a2bb, nd, ns, W2, b2.reshape(1, H2), Wm)
    a3a, a3b = _agg(h2p, h2p, src, dst, zrows)

    eps = jax.random.normal(jax.random.key(42), (N_NODES, D_OUT), jnp.float32)
    eps = jnp.pad(eps, ((0, NPAD - N_NODES), (0, 0)))
    z, zm = _tc3(a3a, a3b, nd, bm.reshape(1, D_OUT), eps)
    return (z[:N_NODES], zm[:N_NODES], zm[:N_NODES])


# speedup vs baseline: 7.8994x; 2.2054x over previous
import jax
import jax.numpy as jnp
from jax import lax
from jax.experimental import pallas as pl
from jax.experimental.pallas import tpu as pltpu
from jax.experimental.pallas import tpu_sc as plsc

N_NODES = 10000
N_EDGES = 320000
D_FEAT = 128
H1 = 256
H2 = 256
D_OUT = 128

NC = 2
NS = 16
EB = 80
NPAD = 10240
RPT = NPAD // NS
CHUNK_ROWS = 80
NCH = RPT // CHUNK_ROWS


def _sc_mesh():
    return plsc.VectorSubcoreMesh(
        core_axis_name="c", subcore_axis_name="s", num_cores=NC, num_subcores=NS
    )


def _zero_acc(zero_hbm, zbuf, acc, s):
    pltpu.sync_copy(zero_hbm, zbuf)

    def z(j, carry):
        pltpu.sync_copy(zbuf, acc.at[pl.ds(s * RPT + j * CHUNK_ROWS, CHUNK_ROWS)])
        return carry

    lax.fori_loop(0, NCH, z, 0)


def _copy_out(acc, obuf, out_hbm, s):
    def co(j, carry):
        r = s * RPT + j * CHUNK_ROWS
        pltpu.sync_copy(acc.at[pl.ds(r, CHUNK_ROWS)], obuf)
        pltpu.sync_copy(obuf, out_hbm.at[pl.ds(r, CHUNK_ROWS)])
        return carry

    lax.fori_loop(0, NCH, co, 0)


def _deg_body(src_hbm, dst_hbm, ones_hbm, zero_hbm, outa, outb,
              didx_a, didx_b, onesv, obuf, acc, isem):
    c = lax.axis_index("c")
    s = lax.axis_index("s")
    _zero_acc(zero_hbm, obuf, acc, s)
    pltpu.sync_copy(ones_hbm, onesv)
    plsc.subcore_barrier()
    epw = N_EDGES // NS
    nb = epw // EB

    def run(idx_hbm):
        def didx_load(i, buf):
            return pltpu.async_copy(idx_hbm.at[pl.ds(s * epw + i * EB, EB)],
                                    buf, isem)

        def wait_didx(buf):
            pltpu.make_async_copy(idx_hbm.at[pl.ds(s * epw, EB)], buf,
                                  isem).wait()

        didx_load(0, didx_a)

        def phase(i, dcur, dnxt):
            @pl.when(i + 1 < nb)
            def _():
                didx_load(i + 1, dnxt)
            wait_didx(dcur)
            pltpu.sync_copy(onesv, acc.at[dcur], add=True)

        def it(i2, carry):
            phase(2 * i2, didx_a, didx_b)
            phase(2 * i2 + 1, didx_b, didx_a)
            return carry

        lax.fori_loop(0, nb // 2, it, 0)

    @pl.when(c == 0)
    def _():
        run(dst_hbm)

    @pl.when(c == 1)
    def _():
        run(src_hbm)

    plsc.subcore_barrier()

    @pl.when(c == 0)
    def _():
        _copy_out(acc, obuf, outa, s)

    @pl.when(c == 1)
    def _():
        _copy_out(acc, obuf, outb, s)


_deg_call = pl.kernel(
    _deg_body,
    out_type=[
        jax.ShapeDtypeStruct((NPAD, 128), jnp.float32),
        jax.ShapeDtypeStruct((NPAD, 128), jnp.float32),
    ],
    mesh=_sc_mesh(),
    scratch_types=[
        pltpu.VMEM((EB,), jnp.int32),
        pltpu.VMEM((EB,), jnp.int32),
        pltpu.VMEM((EB, 128), jnp.float32),
        pltpu.VMEM((CHUNK_ROWS, 128), jnp.float32),
        pltpu.VMEM_SHARED((NPAD, 128), jnp.float32),
        pltpu.SemaphoreType.DMA,
    ],
)


def _agg_body(hsa, hsb, src_hbm, dst_hbm, zero_hbm, outa, outb,
              sidx_all, didx_a, didx_b, rows2, obuf, acc, gsem, isem):
    c = lax.axis_index("c")
    s = lax.axis_index("s")
    _zero_acc(zero_hbm, obuf, acc, s)
    plsc.subcore_barrier()
    epw = N_EDGES // NS // 2
    nb = epw // EB

    def run(hs, ebase):
        base0 = ebase + s * epw
        pltpu.sync_copy(src_hbm.at[pl.ds(base0, epw)], sidx_all)

        def didx_load(i, buf):
            return pltpu.async_copy(dst_hbm.at[pl.ds(base0 + i * EB, EB)],
                                    buf, isem)

        def gather(i, p):
            return pltpu.async_copy(
                hs.at[sidx_all.at[pl.ds(i * EB, EB)]], rows2.at[p], gsem)

        def wait_gather(p):
            pltpu.make_async_copy(hs.at[sidx_all.at[pl.ds(0, EB)]],
                                  rows2.at[p], gsem).wait()

        def wait_didx(buf):
            pltpu.make_async_copy(dst_hbm.at[pl.ds(base0, EB)], buf,
                                  isem).wait()

        didx_load(0, didx_a)
        gather(0, 0)

        def phase(i, p, dcur, dnxt):
            @pl.when(i + 1 < nb)
            def _():
                didx_load(i + 1, dnxt)
                gather(i + 1, 1 - p)
            wait_gather(p)
            wait_didx(dcur)
            pltpu.sync_copy(rows2.at[p], acc.at[dcur], add=True)

        def it(i2, carry):
            phase(2 * i2, 0, didx_a, didx_b)
            phase(2 * i2 + 1, 1, didx_b, didx_a)
            return carry

        lax.fori_loop(0, nb // 2, it, 0)
        phase(nb - 1, 0, didx_a, didx_b)

    @pl.when(c == 0)
    def _():
        run(hsa, 0)

    @pl.when(c == 1)
    def _():
        run(hsb, N_EDGES // 2)

    plsc.subcore_barrier()

    @pl.when(c == 0)
    def _():
        _copy_out(acc, obuf, outa, s)

    @pl.when(c == 1)
    def _():
        _copy_out(acc, obuf, outb, s)


_agg = pl.kernel(
    _agg_body,
    out_type=[
        jax.ShapeDtypeStruct((NPAD, 128), jnp.float32),
        jax.ShapeDtypeStruct((NPAD, 128), jnp.float32),
    ],
    mesh=_sc_mesh(),
    scratch_types=[
        pltpu.VMEM((N_EDGES // NS // 2,), jnp.int32),
        pltpu.VMEM((EB,), jnp.int32),
        pltpu.VMEM((EB,), jnp.int32),
        pltpu.VMEM((2, EB, 128), jnp.float32),
        pltpu.VMEM((CHUNK_ROWS, 128), jnp.float32),
        pltpu.VMEM_SHARED((NPAD, 128), jnp.float32),
        pltpu.SemaphoreType.DMA,
        pltpu.SemaphoreType.DMA,
    ],
)


_RB = 1024
_NB = NPAD // _RB


def _rows(ref, i):
    return ref[pl.ds(i * _RB, _RB), :]


def _tc1_body(aa_ref, ab_ref, nd_ref, ns_ref, w_ref, b_ref, oa_ref, ob_ref):
    i = pl.program_id(0)
    nd = _rows(nd_ref, i)
    agg = (aa_ref[...] + ab_ref[...]) * nd
    h = jnp.tanh(
        jnp.dot(agg, w_ref[...], preferred_element_type=jnp.float32) + b_ref[...]
    )
    h = h * _rows(ns_ref, i)
    oa_ref[...] = h[:, : H1 // 2]
    ob_ref[...] = h[:, H1 // 2:]


def _tc2_body(aa0_ref, aa1_ref, ab0_ref, ab1_ref, nd_ref, ns_ref, w_ref,
              b_ref, wm_ref, o_ref):
    i = pl.program_id(0)
    nd = _rows(nd_ref, i)
    agg = jnp.concatenate(
        [aa0_ref[...] + aa1_ref[...], ab0_ref[...] + ab1_ref[...]], axis=1
    ) * nd
    h = jax.nn.relu(
        jnp.dot(agg, w_ref[...], preferred_element_type=jnp.float32) + b_ref[...]
    )
    h = h * _rows(ns_ref, i)
    o_ref[...] = jnp.dot(h, wm_ref[...], preferred_element_type=jnp.float32)


def _tc3_body(aa_ref, ab_ref, nd_ref, b_ref, eps_ref, z_ref, zm_ref):
    i = pl.program_id(0)
    nd = _rows(nd_ref, i)
    zm = (aa_ref[...] + ab_ref[...]) * nd + b_ref[...]
    zm_ref[...] = zm
    z_ref[...] = zm + jnp.exp(zm) * eps_ref[...]


def _full(shape):
    return pl.BlockSpec(shape, lambda i: (0, 0))


def _blk(cols):
    return pl.BlockSpec((_RB, cols), lambda i: (i, 0))


def _tc1(aa, ab, nd, ns, w, b):
    return pl.pallas_call(
        _tc1_body,
        grid=(_NB,),
        in_specs=[
            _blk(D_FEAT), _blk(D_FEAT),
            _full((NPAD, 1)), _full((NPAD, 1)),
            _full((D_FEAT, H1)), _full((1, H1)),
        ],
        out_specs=[_blk(H1 // 2), _blk(H1 // 2)],
        out_shape=[
            jax.ShapeDtypeStruct((NPAD, H1 // 2), jnp.float32),
            jax.ShapeDtypeStruct((NPAD, H1 // 2), jnp.float32),
        ],
    )(aa, ab, nd, ns, w, b)


def _tc2(aa0, aa1, ab0, ab1, nd, ns, w, b, wm):
    return pl.pallas_call(
        _tc2_body,
        grid=(_NB,),
        in_specs=[
            _blk(H1 // 2), _blk(H1 // 2), _blk(H1 // 2), _blk(H1 // 2),
            _full((NPAD, 1)), _full((NPAD, 1)),
            _full((H1, H2)), _full((1, H2)), _full((H2, D_OUT)),
        ],
        out_specs=_blk(D_OUT),
        out_shape=jax.ShapeDtypeStruct((NPAD, D_OUT), jnp.float32),
    )(aa0, aa1, ab0, ab1, nd, ns, w, b, wm)


def _tc3(aa, ab, nd, b, eps):
    return pl.pallas_call(
        _tc3_body,
        grid=(_NB,),
        in_specs=[
            _blk(D_OUT), _blk(D_OUT),
            _full((NPAD, 1)), _full((1, D_OUT)), _blk(D_OUT),
        ],
        out_specs=[_blk(D_OUT), _blk(D_OUT)],
        out_shape=[
            jax.ShapeDtypeStruct((NPAD, D_OUT), jnp.float32),
            jax.ShapeDtypeStruct((NPAD, D_OUT), jnp.float32),
        ],
    )(aa, ab, nd, b, eps)


def kernel(x, edge_index, W1, b1, W2, b2, Wm, bm):
    src = edge_index[0]
    dst = edge_index[1]

    ones = jnp.ones((EB, 128), jnp.float32)
    zrows = jnp.zeros((CHUNK_ROWS, 128), jnp.float32)

    degi, dego = _deg_call(src, dst, ones, zrows)
    ns = lax.rsqrt(jnp.clip(dego[:, :1], 1.0, None))
    nd = lax.rsqrt(jnp.clip(degi[:, :1], 1.0, None))

    xs = jnp.pad(x * ns[:N_NODES], ((0, NPAD - N_NODES), (0, 0)))
    a1a, a1b = _agg(xs, xs, src, dst, zrows)
    h1a, h1b = _tc1(a1a, a1b, nd, ns, W1, b1.reshape(1, H1))
    a2aa, a2ab = _agg(h1a, h1a, src, dst, zrows)
    a2ba, a2bb = _agg(h1b, h1b, src, dst, zrows)
    h2p = _tc2(a2aa, a2ab, a2ba, a2bb, nd, ns, W2, b2.reshape(1, H2), Wm)
    a3a, a3b = _agg(h2p, h2p, src, dst, zrows)

    eps = jax.random.normal(jax.random.key(42), (N_NODES, D_OUT), jnp.float32)
    eps = jnp.pad(eps, ((0, NPAD - N_NODES), (0, 0)))
    z, zm = _tc3(a3a, a3b, nd, bm.reshape(1, D_OUT), eps)
    return (z[:N_NODES], zm[:N_NODES], zm[:N_NODES])


# speedup vs baseline: 7.9308x; 1.0040x over previous
import jax
import jax.numpy as jnp
from jax import lax
from jax.experimental import pallas as pl
from jax.experimental.pallas import tpu as pltpu
from jax.experimental.pallas import tpu_sc as plsc

N_NODES = 10000
N_EDGES = 320000
D_FEAT = 128
H1 = 256
H2 = 256
D_OUT = 128

NC = 2
NS = 16
EB = 80
NPAD = 10240
RPT = NPAD // NS
CHUNK_ROWS = 80
NCH = RPT // CHUNK_ROWS


def _sc_mesh():
    return plsc.VectorSubcoreMesh(
        core_axis_name="c", subcore_axis_name="s", num_cores=NC, num_subcores=NS
    )


def _zero_acc(zero_hbm, zbuf, acc, s):
    pltpu.sync_copy(zero_hbm, zbuf)

    def z(j, carry):
        pltpu.sync_copy(zbuf, acc.at[pl.ds(s * RPT + j * CHUNK_ROWS, CHUNK_ROWS)])
        return carry

    lax.fori_loop(0, NCH, z, 0)


def _copy_out(acc, obuf, out_hbm, s):
    def co(j, carry):
        r = s * RPT + j * CHUNK_ROWS
        pltpu.sync_copy(acc.at[pl.ds(r, CHUNK_ROWS)], obuf)
        pltpu.sync_copy(obuf, out_hbm.at[pl.ds(r, CHUNK_ROWS)])
        return carry

    lax.fori_loop(0, NCH, co, 0)


def _deg_body(src_hbm, dst_hbm, ones_hbm, zero_hbm, outa, outb,
              didx_a, didx_b, onesv, obuf, acc, isem, ssem):
    c = lax.axis_index("c")
    s = lax.axis_index("s")
    _zero_acc(zero_hbm, obuf, acc, s)
    pltpu.sync_copy(ones_hbm, onesv)
    plsc.subcore_barrier()
    epw = N_EDGES // NS
    nb = epw // EB

    def run(idx_hbm):
        def didx_load(i, buf):
            return pltpu.async_copy(idx_hbm.at[pl.ds(s * epw + i * EB, EB)],
                                    buf, isem)

        def wait_didx(buf):
            pltpu.make_async_copy(idx_hbm.at[pl.ds(s * epw, EB)], buf,
                                  isem).wait()

        def wait_scatter():
            pltpu.make_async_copy(onesv, acc.at[didx_a], ssem).wait()

        didx_load(0, didx_a)

        def phase(i, dcur, dnxt):
            @pl.when(i > 0)
            def _():
                wait_scatter()

            @pl.when(i + 1 < nb)
            def _():
                didx_load(i + 1, dnxt)
            wait_didx(dcur)
            pltpu.async_copy(onesv, acc.at[dcur], ssem, add=True)

        def it(i2, carry):
            phase(2 * i2, didx_a, didx_b)
            phase(2 * i2 + 1, didx_b, didx_a)
            return carry

        lax.fori_loop(0, nb // 2, it, 0)
        wait_scatter()

    @pl.when(c == 0)
    def _():
        run(dst_hbm)

    @pl.when(c == 1)
    def _():
        run(src_hbm)

    plsc.subcore_barrier()

    @pl.when(c == 0)
    def _():
        _copy_out(acc, obuf, outa, s)

    @pl.when(c == 1)
    def _():
        _copy_out(acc, obuf, outb, s)


_deg_call = pl.kernel(
    _deg_body,
    out_type=[
        jax.ShapeDtypeStruct((NPAD, 128), jnp.float32),
        jax.ShapeDtypeStruct((NPAD, 128), jnp.float32),
    ],
    mesh=_sc_mesh(),
    scratch_types=[
        pltpu.VMEM((EB,), jnp.int32),
        pltpu.VMEM((EB,), jnp.int32),
        pltpu.VMEM((EB, 128), jnp.float32),
        pltpu.VMEM((CHUNK_ROWS, 128), jnp.float32),
        pltpu.VMEM_SHARED((NPAD, 128), jnp.float32),
        pltpu.SemaphoreType.DMA,
        pltpu.SemaphoreType.DMA,
    ],
)


def _agg_body(hsa, hsb, src_hbm, dst_hbm, zero_hbm, outa, outb,
              sidx_all, didx_a, didx_b, rows2, obuf, acc, gsem, isem, ssem):
    c = lax.axis_index("c")
    s = lax.axis_index("s")
    _zero_acc(zero_hbm, obuf, acc, s)
    plsc.subcore_barrier()
    epw = N_EDGES // NS // 2
    nb = epw // EB

    def run(hs, ebase):
        base0 = ebase + s * epw
        pltpu.sync_copy(src_hbm.at[pl.ds(base0, epw)], sidx_all)

        def didx_load(i, buf):
            return pltpu.async_copy(dst_hbm.at[pl.ds(base0 + i * EB, EB)],
                                    buf, isem)

        def gather(i, p):
            return pltpu.async_copy(
                hs.at[sidx_all.at[pl.ds(i * EB, EB)]], rows2.at[p], gsem)

        def wait_gather(p):
            pltpu.make_async_copy(hs.at[sidx_all.at[pl.ds(0, EB)]],
                                  rows2.at[p], gsem).wait()

        def wait_didx(buf):
            pltpu.make_async_copy(dst_hbm.at[pl.ds(base0, EB)], buf,
                                  isem).wait()

        def wait_scatter():
            pltpu.make_async_copy(rows2.at[0], acc.at[didx_a], ssem).wait()

        didx_load(0, didx_a)
        gather(0, 0)

        def phase(i, p, dcur, dnxt):
            @pl.when(i > 0)
            def _():
                wait_scatter()

            @pl.when(i + 1 < nb)
            def _():
                didx_load(i + 1, dnxt)
                gather(i + 1, 1 - p)
            wait_gather(p)
            wait_didx(dcur)
            pltpu.async_copy(rows2.at[p], acc.at[dcur], ssem, add=True)

        def it(i2, carry):
            phase(2 * i2, 0, didx_a, didx_b)
            phase(2 * i2 + 1, 1, didx_b, didx_a)
            return carry

        lax.fori_loop(0, nb // 2, it, 0)
        phase(nb - 1, 0, didx_a, didx_b)
        wait_scatter()

    @pl.when(c == 0)
    def _():
        run(hsa, 0)

    @pl.when(c == 1)
    def _():
        run(hsb, N_EDGES // 2)

    plsc.subcore_barrier()

    @pl.when(c == 0)
    def _():
        _copy_out(acc, obuf, outa, s)

    @pl.when(c == 1)
    def _():
        _copy_out(acc, obuf, outb, s)


_agg = pl.kernel(
    _agg_body,
    out_type=[
        jax.ShapeDtypeStruct((NPAD, 128), jnp.float32),
        jax.ShapeDtypeStruct((NPAD, 128), jnp.float32),
    ],
    mesh=_sc_mesh(),
    scratch_types=[
        pltpu.VMEM((N_EDGES // NS // 2,), jnp.int32),
        pltpu.VMEM((EB,), jnp.int32),
        pltpu.VMEM((EB,), jnp.int32),
        pltpu.VMEM((2, EB, 128), jnp.float32),
        pltpu.VMEM((CHUNK_ROWS, 128), jnp.float32),
        pltpu.VMEM_SHARED((NPAD, 128), jnp.float32),
        pltpu.SemaphoreType.DMA,
        pltpu.SemaphoreType.DMA,
        pltpu.SemaphoreType.DMA,
    ],
)


_RB = 1024
_NB = NPAD // _RB


def _rows(ref, i):
    return ref[pl.ds(i * _RB, _RB), :]


def _tc1_body(aa_ref, ab_ref, nd_ref, ns_ref, w_ref, b_ref, oa_ref, ob_ref):
    i = pl.program_id(0)
    nd = _rows(nd_ref, i)
    agg = (aa_ref[...] + ab_ref[...]) * nd
    h = jnp.tanh(
        jnp.dot(agg, w_ref[...], preferred_element_type=jnp.float32) + b_ref[...]
    )
    h = h * _rows(ns_ref, i)
    oa_ref[...] = h[:, : H1 // 2]
    ob_ref[...] = h[:, H1 // 2:]


def _tc2_body(aa0_ref, aa1_ref, ab0_ref, ab1_ref, nd_ref, ns_ref, w_ref,
              b_ref, wm_ref, o_ref):
    i = pl.program_id(0)
    nd = _rows(nd_ref, i)
    agg = jnp.concatenate(
        [aa0_ref[...] + aa1_ref[...], ab0_ref[...] + ab1_ref[...]], axis=1
    ) * nd
    h = jax.nn.relu(
        jnp.dot(agg, w_ref[...], preferred_element_type=jnp.float32) + b_ref[...]
    )
    h = h * _rows(ns_ref, i)
    o_ref[...] = jnp.dot(h, wm_ref[...], preferred_element_type=jnp.float32)


def _tc3_body(aa_ref, ab_ref, nd_ref, b_ref, eps_ref, z_ref, zm_ref):
    i = pl.program_id(0)
    nd = nd_ref[pl.ds(i * 1000, 1000), :]
    zm = (aa_ref[...] + ab_ref[...]) * nd + b_ref[...]
    zm_ref[...] = zm
    z_ref[...] = zm + jnp.exp(zm) * eps_ref[...]


def _full(shape):
    return pl.BlockSpec(shape, lambda i: (0, 0))


def _blk(cols):
    return pl.BlockSpec((_RB, cols), lambda i: (i, 0))


def _tc1(aa, ab, nd, ns, w, b):
    return pl.pallas_call(
        _tc1_body,
        grid=(_NB,),
        in_specs=[
            _blk(D_FEAT), _blk(D_FEAT),
            _full((NPAD, 1)), _full((NPAD, 1)),
            _full((D_FEAT, H1)), _full((1, H1)),
        ],
        out_specs=[_blk(H1 // 2), _blk(H1 // 2)],
        out_shape=[
            jax.ShapeDtypeStruct((NPAD, H1 // 2), jnp.float32),
            jax.ShapeDtypeStruct((NPAD, H1 // 2), jnp.float32),
        ],
    )(aa, ab, nd, ns, w, b)


def _tc2(aa0, aa1, ab0, ab1, nd, ns, w, b, wm):
    return pl.pallas_call(
        _tc2_body,
        grid=(_NB,),
        in_specs=[
            _blk(H1 // 2), _blk(H1 // 2), _blk(H1 // 2), _blk(H1 // 2),
            _full((NPAD, 1)), _full((NPAD, 1)),
            _full((H1, H2)), _full((1, H2)), _full((H2, D_OUT)),
        ],
        out_specs=_blk(D_OUT),
        out_shape=jax.ShapeDtypeStruct((NPAD, D_OUT), jnp.float32),
    )(aa0, aa1, ab0, ab1, nd, ns, w, b, wm)


def _tc3(aa, ab, nd, b, eps):
    blk = pl.BlockSpec((1000, D_OUT), lambda i: (i, 0))
    return pl.pallas_call(
        _tc3_body,
        grid=(N_NODES // 1000,),
        in_specs=[
            blk, blk,
            _full((NPAD, 1)), _full((1, D_OUT)), blk,
        ],
        out_specs=[blk, blk],
        out_shape=[
            jax.ShapeDtypeStruct((N_NODES, D_OUT), jnp.float32),
            jax.ShapeDtypeStruct((N_NODES, D_OUT), jnp.float32),
        ],
    )(aa, ab, nd, b, eps)


def kernel(x, edge_index, W1, b1, W2, b2, Wm, bm):
    src = edge_index[0]
    dst = edge_index[1]

    ones = jnp.ones((EB, 128), jnp.float32)
    zrows = jnp.zeros((CHUNK_ROWS, 128), jnp.float32)

    degi, dego = _deg_call(src, dst, ones, zrows)
    ns = lax.rsqrt(jnp.clip(dego[:, :1], 1.0, None))
    nd = lax.rsqrt(jnp.clip(degi[:, :1], 1.0, None))

    xs = jnp.pad(x * ns[:N_NODES], ((0, NPAD - N_NODES), (0, 0)))
    a1a, a1b = _agg(xs, xs, src, dst, zrows)
    h1a, h1b = _tc1(a1a, a1b, nd, ns, W1, b1.reshape(1, H1))
    a2aa, a2ab = _agg(h1a, h1a, src, dst, zrows)
    a2ba, a2bb = _agg(h1b, h1b, src, dst, zrows)
    h2p = _tc2(a2aa, a2ab, a2ba, a2bb, nd, ns, W2, b2.reshape(1, H2), Wm)
    a3a, a3b = _agg(h2p, h2p, src, dst, zrows)

    eps = jax.random.normal(jax.random.key(42), (N_NODES, D_OUT), jnp.float32)
    z, zm = _tc3(a3a, a3b, nd, bm.reshape(1, D_OUT), eps)
    return (z, zm, zm)


# speedup vs baseline: 8.6039x; 1.0849x over previous
import jax
import jax.numpy as jnp
from jax import lax
from jax.experimental import pallas as pl
from jax.experimental.pallas import tpu as pltpu
from jax.experimental.pallas import tpu_sc as plsc

N_NODES = 10000
N_EDGES = 320000
D_FEAT = 128
H1 = 256
H2 = 256
D_OUT = 128

NC = 2
NS = 16
EB = 128
NPAD = 10240
RPT = NPAD // NS
CHUNK_ROWS = 16
NCH = RPT // CHUNK_ROWS


def _sc_mesh():
    return plsc.VectorSubcoreMesh(
        core_axis_name="c", subcore_axis_name="s", num_cores=NC, num_subcores=NS
    )


def _zero_acc(zero_hbm, obuf2, acc, s, wsem):
    pltpu.sync_copy(zero_hbm, obuf2.at[0])

    def z(j, carry):
        pltpu.async_copy(obuf2.at[0],
                         acc.at[pl.ds(s * RPT + j * CHUNK_ROWS, CHUNK_ROWS)], wsem)
        return carry

    lax.fori_loop(0, NCH, z, 0)

    def zw(j, carry):
        pltpu.make_async_copy(
            obuf2.at[0], acc.at[pl.ds(s * RPT, CHUNK_ROWS)], wsem).wait()
        return carry

    lax.fori_loop(0, NCH, zw, 0)


def _copy_out(acc, obuf2, out_hbm, s, wsem):
    def wait_w(j):
        pltpu.make_async_copy(
            obuf2.at[0], out_hbm.at[pl.ds(s * RPT, CHUNK_ROWS)], wsem).wait()

    def co(j, p):
        r = s * RPT + j * CHUNK_ROWS
        pltpu.sync_copy(acc.at[pl.ds(r, CHUNK_ROWS)], obuf2.at[p])

        @pl.when(j > 1)
        def _():
            wait_w(j - 2)
        pltpu.async_copy(obuf2.at[p], out_hbm.at[pl.ds(r, CHUNK_ROWS)], wsem)

    def it(j2, carry):
        co(2 * j2, 0)
        co(2 * j2 + 1, 1)
        return carry

    lax.fori_loop(0, NCH // 2, it, 0)
    wait_w(NCH - 2)
    wait_w(NCH - 1)


def _deg_body(src_hbm, dst_hbm, ones_hbm, zero_hbm, outa, outb,
              didx_a, didx_b, onesv, obuf2, acc, isem, ssem, wsem):
    c = lax.axis_index("c")
    s = lax.axis_index("s")
    _zero_acc(zero_hbm, obuf2, acc, s, wsem)
    pltpu.sync_copy(ones_hbm, onesv)
    plsc.subcore_barrier()
    nb = 156 + jnp.where(s < 4, 1, 0)
    base0 = (156 * s + jnp.minimum(s, 4)) * EB

    def run(idx_hbm):
        def didx_load(i, buf):
            return pltpu.async_copy(idx_hbm.at[pl.ds(base0 + i * EB, EB)],
                                    buf, isem)

        def wait_didx(buf):
            pltpu.make_async_copy(idx_hbm.at[pl.ds(0, EB)], buf, isem).wait()

        def wait_scatter():
            pltpu.make_async_copy(onesv, acc.at[didx_a], ssem).wait()

        didx_load(0, didx_a)

        def phase(i, dcur, dnxt, prefetch=True):
            @pl.when(i > 0)
            def _():
                wait_scatter()

            if prefetch:
                @pl.when(i + 1 < nb)
                def _():
                    didx_load(i + 1, dnxt)
            wait_didx(dcur)
            pltpu.async_copy(onesv, acc.at[dcur], ssem, add=True)

        def it(i2, carry):
            phase(2 * i2, didx_a, didx_b)
            phase(2 * i2 + 1, didx_b, didx_a)
            return carry

        lax.fori_loop(0, 156 // 2, it, 0)

        @pl.when(s < 4)
        def _():
            phase(156, didx_a, didx_b, prefetch=False)
        wait_scatter()

    @pl.when(c == 0)
    def _():
        run(dst_hbm)

    @pl.when(c == 1)
    def _():
        run(src_hbm)

    plsc.subcore_barrier()

    @pl.when(c == 0)
    def _():
        _copy_out(acc, obuf2, outa, s, wsem)

    @pl.when(c == 1)
    def _():
        _copy_out(acc, obuf2, outb, s, wsem)


_deg_call = pl.kernel(
    _deg_body,
    out_type=[
        jax.ShapeDtypeStruct((NPAD, 128), jnp.float32),
        jax.ShapeDtypeStruct((NPAD, 128), jnp.float32),
    ],
    mesh=_sc_mesh(),
    scratch_types=[
        pltpu.VMEM((EB,), jnp.int32),
        pltpu.VMEM((EB,), jnp.int32),
        pltpu.VMEM((EB, 128), jnp.float32),
        pltpu.VMEM((2, CHUNK_ROWS, 128), jnp.float32),
        pltpu.VMEM_SHARED((NPAD, 128), jnp.float32),
        pltpu.SemaphoreType.DMA,
        pltpu.SemaphoreType.DMA,
        pltpu.SemaphoreType.DMA,
    ],
)


def _agg_body(hsa, hsb, src_hbm, dst_hbm, zero_hbm, outa, outb,
              sidx_all, didx_a, didx_b, rows2, obuf2, acc, gsem, isem, ssem,
              wsem):
    c = lax.axis_index("c")
    s = lax.axis_index("s")
    _zero_acc(zero_hbm, obuf2, acc, s, wsem)
    plsc.subcore_barrier()
    nb = 78 + jnp.where(s < 2, 1, 0)

    def run(hs, ebase):
        base0 = ebase + (78 * s + jnp.minimum(s, 2)) * EB
        pltpu.sync_copy(src_hbm.at[pl.ds(base0, 78 * EB)],
                        sidx_all.at[pl.ds(0, 78 * EB)])

        @pl.when(s < 2)
        def _():
            pltpu.sync_copy(src_hbm.at[pl.ds(base0 + 78 * EB, EB)],
                            sidx_all.at[pl.ds(78 * EB, EB)])

        def didx_load(i, buf):
            return pltpu.async_copy(dst_hbm.at[pl.ds(base0 + i * EB, EB)],
                                    buf, isem)

        def gather(i, p):
            return pltpu.async_copy(
                hs.at[sidx_all.at[pl.ds(i * EB, EB)]], rows2.at[p], gsem)

        def wait_gather(p):
            pltpu.make_async_copy(hs.at[sidx_all.at[pl.ds(0, EB)]],
                                  rows2.at[p], gsem).wait()

        def wait_didx(buf):
            pltpu.make_async_copy(dst_hbm.at[pl.ds(0, EB)], buf, isem).wait()

        def wait_scatter():
            pltpu.make_async_copy(rows2.at[0], acc.at[didx_a], ssem).wait()

        didx_load(0, didx_a)
        gather(0, 0)

        def phase(i, p, dcur, dnxt, prefetch=True):
            @pl.when(i > 0)
            def _():
                wait_scatter()

            if prefetch:
                @pl.when(i + 1 < nb)
                def _():
                    didx_load(i + 1, dnxt)
                    gather(i + 1, 1 - p)
            wait_gather(p)
            wait_didx(dcur)
            pltpu.async_copy(rows2.at[p], acc.at[dcur], ssem, add=True)

        def it(i2, carry):
            phase(2 * i2, 0, didx_a, didx_b)
            phase(2 * i2 + 1, 1, didx_b, didx_a)
            return carry

        lax.fori_loop(0, 78 // 2, it, 0)

        @pl.when(s < 2)
        def _():
            phase(78, 0, didx_a, didx_b, prefetch=False)
        wait_scatter()

    @pl.when(c == 0)
    def _():
        run(hsa, 0)

    @pl.when(c == 1)
    def _():
        run(hsb, N_EDGES // 2)

    plsc.subcore_barrier()

    @pl.when(c == 0)
    def _():
        _copy_out(acc, obuf2, outa, s, wsem)

    @pl.when(c == 1)
    def _():
        _copy_out(acc, obuf2, outb, s, wsem)


_agg = pl.kernel(
    _agg_body,
    out_type=[
        jax.ShapeDtypeStruct((NPAD, 128), jnp.float32),
        jax.ShapeDtypeStruct((NPAD, 128), jnp.float32),
    ],
    mesh=_sc_mesh(),
    scratch_types=[
        pltpu.VMEM((79 * EB,), jnp.int32),
        pltpu.VMEM((EB,), jnp.int32),
        pltpu.VMEM((EB,), jnp.int32),
        pltpu.VMEM((2, EB, 128), jnp.float32),
        pltpu.VMEM((2, CHUNK_ROWS, 128), jnp.float32),
        pltpu.VMEM_SHARED((NPAD, 128), jnp.float32),
        pltpu.SemaphoreType.DMA,
        pltpu.SemaphoreType.DMA,
        pltpu.SemaphoreType.DMA,
        pltpu.SemaphoreType.DMA,
    ],
)


_RB = 1024
_NB = NPAD // _RB


def _rows(ref, i):
    return ref[pl.ds(i * _RB, _RB), :]


def _tc1_body(aa_ref, ab_ref, nd_ref, ns_ref, w_ref, b_ref, oa_ref, ob_ref):
    i = pl.program_id(0)
    nd = _rows(nd_ref, i)
    agg = (aa_ref[...] + ab_ref[...]) * nd
    h = jnp.tanh(
        jnp.dot(agg, w_ref[...], preferred_element_type=jnp.float32) + b_ref[...]
    )
    h = h * _rows(ns_ref, i)
    oa_ref[...] = h[:, : H1 // 2]
    ob_ref[...] = h[:, H1 // 2:]


def _tc2_body(aa0_ref, aa1_ref, ab0_ref, ab1_ref, nd_ref, ns_ref, w_ref,
              b_ref, wm_ref, o_ref):
    i = pl.program_id(0)
    nd = _rows(nd_ref, i)
    agg = jnp.concatenate(
        [aa0_ref[...] + aa1_ref[...], ab0_ref[...] + ab1_ref[...]], axis=1
    ) * nd
    h = jax.nn.relu(
        jnp.dot(agg, w_ref[...], preferred_element_type=jnp.float32) + b_ref[...]
    )
    h = h * _rows(ns_ref, i)
    o_ref[...] = jnp.dot(h, wm_ref[...], preferred_element_type=jnp.float32)


def _tc3_body(aa_ref, ab_ref, nd_ref, b_ref, eps_ref, z_ref, zm_ref):
    i = pl.program_id(0)
    nd = nd_ref[pl.ds(i * 1000, 1000), :]
    zm = (aa_ref[...] + ab_ref[...]) * nd + b_ref[...]
    zm_ref[...] = zm
    z_ref[...] = zm + jnp.exp(zm) * eps_ref[...]


def _full(shape):
    return pl.BlockSpec(shape, lambda i: (0, 0))


def _blk(cols):
    return pl.BlockSpec((_RB, cols), lambda i: (i, 0))


def _tc1(aa, ab, nd, ns, w, b):
    return pl.pallas_call(
        _tc1_body,
        grid=(_NB,),
        in_specs=[
            _blk(D_FEAT), _blk(D_FEAT),
            _full((NPAD, 1)), _full((NPAD, 1)),
            _full((D_FEAT, H1)), _full((1, H1)),
        ],
        out_specs=[_blk(H1 // 2), _blk(H1 // 2)],
        out_shape=[
            jax.ShapeDtypeStruct((NPAD, H1 // 2), jnp.float32),
            jax.ShapeDtypeStruct((NPAD, H1 // 2), jnp.float32),
        ],
    )(aa, ab, nd, ns, w, b)


def _tc2(aa0, aa1, ab0, ab1, nd, ns, w, b, wm):
    return pl.pallas_call(
        _tc2_body,
        grid=(_NB,),
        in_specs=[
            _blk(H1 // 2), _blk(H1 // 2), _blk(H1 // 2), _blk(H1 // 2),
            _full((NPAD, 1)), _full((NPAD, 1)),
            _full((H1, H2)), _full((1, H2)), _full((H2, D_OUT)),
        ],
        out_specs=_blk(D_OUT),
        out_shape=jax.ShapeDtypeStruct((NPAD, D_OUT), jnp.float32),
    )(aa0, aa1, ab0, ab1, nd, ns, w, b, wm)


def _tc3(aa, ab, nd, b, eps):
    blk = pl.BlockSpec((1000, D_OUT), lambda i: (i, 0))
    return pl.pallas_call(
        _tc3_body,
        grid=(N_NODES // 1000,),
        in_specs=[
            blk, blk,
            _full((NPAD, 1)), _full((1, D_OUT)), blk,
        ],
        out_specs=[blk, blk],
        out_shape=[
            jax.ShapeDtypeStruct((N_NODES, D_OUT), jnp.float32),
            jax.ShapeDtypeStruct((N_NODES, D_OUT), jnp.float32),
        ],
    )(aa, ab, nd, b, eps)


def kernel(x, edge_index, W1, b1, W2, b2, Wm, bm):
    src = edge_index[0]
    dst = edge_index[1]

    ones = jnp.ones((EB, 128), jnp.float32)
    zrows = jnp.zeros((CHUNK_ROWS, 128), jnp.float32)

    degi, dego = _deg_call(src, dst, ones, zrows)
    ns = lax.rsqrt(jnp.clip(dego[:, :1], 1.0, None))
    nd = lax.rsqrt(jnp.clip(degi[:, :1], 1.0, None))

    xs = jnp.pad(x * ns[:N_NODES], ((0, NPAD - N_NODES), (0, 0)))
    a1a, a1b = _agg(xs, xs, src, dst, zrows)
    h1a, h1b = _tc1(a1a, a1b, nd, ns, W1, b1.reshape(1, H1))
    a2aa, a2ab = _agg(h1a, h1a, src, dst, zrows)
    a2ba, a2bb = _agg(h1b, h1b, src, dst, zrows)
    h2p = _tc2(a2aa, a2ab, a2ba, a2bb, nd, ns, W2, b2.reshape(1, H2), Wm)
    a3a, a3b = _agg(h2p, h2p, src, dst, zrows)

    eps = jax.random.normal(jax.random.key(42), (N_NODES, D_OUT), jnp.float32)
    z, zm = _tc3(a3a, a3b, nd, bm.reshape(1, D_OUT), eps)
    return (z, zm, zm)
